# baseline (speedup 1.0000x reference)
import functools

import jax
import jax.numpy as jnp
from jax import lax
from jax.experimental import pallas as pl
from jax.experimental.pallas import tpu as pltpu
from jax.experimental.pallas import tpu_sc as plsc

_NUM_CORES = 2
_NUM_SUBCORES = 16
_NUM_WORKERS = _NUM_CORES * _NUM_SUBCORES

_STAGES = ((0, 1024), (1024, 512), (1536, 512))
_C = 16
_R = 512


def _gather_body(B, S, W, off, ids_hbm, tok_hbm, out_hbm, idx_all,
                 tok_a, tok_b, tok_c, tok_d,
                 sem_g_a, sem_g_b, sem_g_c, sem_g_d,
                 sem_o_a, sem_o_b, sem_o_c, sem_o_d):
  rpw = B * W // _NUM_WORKERS
  nchunks = rpw // _C
  wpb = W // rpw
  wid = lax.axis_index("s") * _NUM_CORES + lax.axis_index("c")
  base = wid * rpw
  b_idx = wid // wpb
  src = b_idx * S + off + (wid % wpb) * rpw
  pltpu.sync_copy(ids_hbm.at[pl.ds(src, rpw)], idx_all)

  toks = [tok_a, tok_b, tok_c, tok_d]
  sem_gs = [sem_g_a, sem_g_b, sem_g_c, sem_g_d]
  sem_os = [sem_o_a, sem_o_b, sem_o_c, sem_o_d]
  nbuf = len(toks)

  def start_gather(k):
    return pltpu.async_copy(
        tok_hbm.at[idx_all.at[pl.ds(k * _C, _C)]], toks[k % nbuf],
        sem_gs[k % nbuf])

  gcp = [None] * nchunks
  ocp = [None] * nchunks
  for k in range(min(nbuf - 1, nchunks)):
    gcp[k] = start_gather(k)
  for k in range(nchunks):
    if k + nbuf - 1 < nchunks:
      if k >= 1:
        ocp[k - 1].wait()
      gcp[k + nbuf - 1] = start_gather(k + nbuf - 1)
    gcp[k].wait()
    ocp[k] = pltpu.async_copy(
        toks[k % nbuf], out_hbm.at[pl.ds(base + k * _C, _C)],
        sem_os[k % nbuf])
  for k in range(max(nchunks - nbuf, 0), nchunks):
    ocp[k].wait()


def _sc_gather(off, W, ids_flat, token_table, B, S):
  D = token_table.shape[1]
  mesh = plsc.VectorSubcoreMesh(core_axis_name="c", subcore_axis_name="s")
  return pl.kernel(
      functools.partial(_gather_body, B, S, W, off),
      out_type=jax.ShapeDtypeStruct((B * W, D), jnp.float32),
      mesh=mesh,
      scratch_types=[
          pltpu.VMEM((B * W // _NUM_WORKERS,), jnp.int32),
          pltpu.VMEM((_C, D), jnp.float32),
          pltpu.VMEM((_C, D), jnp.float32),
          pltpu.VMEM((_C, D), jnp.float32),
          pltpu.VMEM((_C, D), jnp.float32),
          pltpu.SemaphoreType.DMA,
          pltpu.SemaphoreType.DMA,
          pltpu.SemaphoreType.DMA,
          pltpu.SemaphoreType.DMA,
          pltpu.SemaphoreType.DMA,
          pltpu.SemaphoreType.DMA,
          pltpu.SemaphoreType.DMA,
          pltpu.SemaphoreType.DMA,
      ],
  )(ids_flat, token_table)


def _add_body(tids_ref, *refs):
  g_ref, pos_ref, tt_ref, out_ref = refs[-4:]
  b = pl.program_id(1)
  tid = tids_ref[b]
  trow = tt_ref[pl.ds(tid, 1), :]
  out_ref[...] = g_ref[...] + pos_ref[...] + trow


def _tc_add(off, W, acc, g_j, pos_table, task_table, task_ids, B, S, D):
  hpb = W // _R
  ob = off // _R
  acc_spec = [] if acc is None else [pl.BlockSpec(memory_space=pl.ANY)]
  acc_arg = () if acc is None else (acc,)
  grid_spec = pltpu.PrefetchScalarGridSpec(
      num_scalar_prefetch=1,
      grid=(hpb, B),
      in_specs=acc_spec + [
          pl.BlockSpec((_R, D), lambda h, b, t: (b * hpb + h, 0)),
          pl.BlockSpec((_R, D), lambda h, b, t: (ob + h, 0)),
          pl.BlockSpec((task_table.shape[0], D), lambda h, b, t: (0, 0)),
      ],
      out_specs=pl.BlockSpec(
          (_R, D),
          lambda h, b, t: (b * (S // _R) + ob + h, 0)),
  )
  return pl.pallas_call(
      _add_body,
      grid_spec=grid_spec,
      out_shape=jax.ShapeDtypeStruct((B * S, D), jnp.float32),
      input_output_aliases={1: 0} if acc is not None else {},
  )(task_ids, *acc_arg, g_j, pos_table, task_table)


@jax.jit
def kernel(input_ids, task_ids, token_table, pos_table, task_table):
  B, S = input_ids.shape
  V, D = token_table.shape

  ids = jnp.asarray(input_ids, jnp.int32).reshape(B * S)
  tids = jnp.asarray(task_ids, jnp.int32)

  gs = [_sc_gather(off, W, ids, token_table, B, S) for off, W in _STAGES]

  acc = None
  for (off, W), g in zip(_STAGES, gs):
    acc = _tc_add(off, W, acc, g, pos_table, task_table, tids, B, S, D)
  return acc.reshape(B, S, D)


# speedup vs baseline: 1.0216x; 1.0216x over previous
import functools

import jax
import jax.numpy as jnp
from jax import lax
from jax.experimental import pallas as pl
from jax.experimental.pallas import tpu as pltpu
from jax.experimental.pallas import tpu_sc as plsc

_NUM_CORES = 2
_NUM_SUBCORES = 16
_NUM_WORKERS = _NUM_CORES * _NUM_SUBCORES

_H = 2
_C = 16
_R = 512


def _gather_body(B, S, W, j, ids_hbm, tok_hbm, out_hbm, idx_all,
                 tok_a, tok_b, tok_c, tok_d,
                 sem_g_a, sem_g_b, sem_g_c, sem_g_d,
                 sem_o_a, sem_o_b, sem_o_c, sem_o_d):
  rpw = B * W // _NUM_WORKERS
  nchunks = rpw // _C
  wpb = W // rpw
  wid = lax.axis_index("s") * _NUM_CORES + lax.axis_index("c")
  base = wid * rpw
  b_idx = wid // wpb
  src = b_idx * S + j * W + (wid % wpb) * rpw
  pltpu.sync_copy(ids_hbm.at[pl.ds(src, rpw)], idx_all)

  toks = [tok_a, tok_b, tok_c, tok_d]
  sem_gs = [sem_g_a, sem_g_b, sem_g_c, sem_g_d]
  sem_os = [sem_o_a, sem_o_b, sem_o_c, sem_o_d]
  nbuf = len(toks)

  def start_gather(k):
    return pltpu.async_copy(
        tok_hbm.at[idx_all.at[pl.ds(k * _C, _C)]], toks[k % nbuf],
        sem_gs[k % nbuf])

  gcp = [None] * nchunks
  ocp = [None] * nchunks
  for k in range(min(nbuf - 1, nchunks)):
    gcp[k] = start_gather(k)
  for k in range(nchunks):
    if k + nbuf - 1 < nchunks:
      if k >= 1:
        ocp[k - 1].wait()
      gcp[k + nbuf - 1] = start_gather(k + nbuf - 1)
    gcp[k].wait()
    ocp[k] = pltpu.async_copy(
        toks[k % nbuf], out_hbm.at[pl.ds(base + k * _C, _C)],
        sem_os[k % nbuf])
  for k in range(max(nchunks - nbuf, 0), nchunks):
    ocp[k].wait()


def _sc_gather(j, ids_flat, token_table, B, S):
  W = S // _H
  D = token_table.shape[1]
  mesh = plsc.VectorSubcoreMesh(core_axis_name="c", subcore_axis_name="s")
  return pl.kernel(
      functools.partial(_gather_body, B, S, W, j),
      out_type=jax.ShapeDtypeStruct((B * W, D), jnp.float32),
      mesh=mesh,
      scratch_types=[
          pltpu.VMEM((B * W // _NUM_WORKERS,), jnp.int32),
          pltpu.VMEM((_C, D), jnp.float32),
          pltpu.VMEM((_C, D), jnp.float32),
          pltpu.VMEM((_C, D), jnp.float32),
          pltpu.VMEM((_C, D), jnp.float32),
          pltpu.SemaphoreType.DMA,
          pltpu.SemaphoreType.DMA,
          pltpu.SemaphoreType.DMA,
          pltpu.SemaphoreType.DMA,
          pltpu.SemaphoreType.DMA,
          pltpu.SemaphoreType.DMA,
          pltpu.SemaphoreType.DMA,
          pltpu.SemaphoreType.DMA,
      ],
  )(ids_flat, token_table)


def _add_body(tids_ref, *refs):
  g_ref, pos_ref, tt_ref, out_ref = refs[-4:]
  b = pl.program_id(1)
  tid = tids_ref[b]
  trow = tt_ref[pl.ds(tid, 1), :]
  out_ref[...] = g_ref[...] + pos_ref[...] + trow


def _tc_add(j, acc, g_j, pos_table, task_table, task_ids, B, S, D):
  W = S // _H
  hpb = W // _R
  acc_spec = [] if acc is None else [pl.BlockSpec(memory_space=pl.ANY)]
  acc_arg = () if acc is None else (acc,)
  grid_spec = pltpu.PrefetchScalarGridSpec(
      num_scalar_prefetch=1,
      grid=(hpb, B),
      in_specs=acc_spec + [
          pl.BlockSpec((_R, D), lambda h, b, t: (b * hpb + h, 0)),
          pl.BlockSpec((_R, D), lambda h, b, t: (j * hpb + h, 0)),
          pl.BlockSpec((task_table.shape[0], D), lambda h, b, t: (0, 0)),
      ],
      out_specs=pl.BlockSpec(
          (_R, D),
          lambda h, b, t: (b * (S // _R) + j * hpb + h, 0)),
  )
  return pl.pallas_call(
      _add_body,
      grid_spec=grid_spec,
      out_shape=jax.ShapeDtypeStruct((B * S, D), jnp.float32),
      input_output_aliases={1: 0} if acc is not None else {},
  )(task_ids, *acc_arg, g_j, pos_table, task_table)


@jax.jit
def kernel(input_ids, task_ids, token_table, pos_table, task_table):
  B, S = input_ids.shape
  V, D = token_table.shape

  ids = jnp.asarray(input_ids, jnp.int32).reshape(B * S)
  tids = jnp.asarray(task_ids, jnp.int32)

  gs = [_sc_gather(j, ids, token_table, B, S) for j in range(_H)]

  acc = None
  for j in range(_H):
    acc = _tc_add(j, acc, gs[j], pos_table, task_table, tids, B, S, D)
  return acc.reshape(B, S, D)
